# baseline (speedup 1.0000x reference)
import functools

import jax
import jax.numpy as jnp
from jax import lax
from jax.experimental import pallas as pl
from jax.experimental.pallas import tpu as pltpu
from jax.experimental.pallas import tpu_sc as plsc

N = 10000
E = 320000
D = 128
H = 2 * D
EPS = 1e-7
BN_EPS = 1e-5

NCORE = 2
NSUB = 16
DH = D // NCORE
ROWS = 632
NPAD = ROWS * NSUB
ZR = ROWS // 8
B = 80
EPT = E // NSUB
NB = 256
CH = 32
NCH = NB // CH


def _sc_agg_body(x_hbm, src_hbm, dst_hbm, acc_hbm, acc_sh, sidx, didx,
                 gbuf0, gbuf1, sbuf0, sbuf1,
                 gsem0, gsem1, ssem0, ssem1):
    c = lax.axis_index("c")
    s = lax.axis_index("s")
    base_r = s * ROWS
    gbufs = (gbuf0, gbuf1)
    sbufs = (sbuf0, sbuf1)
    gsems = (gsem0, gsem1)
    ssems = (ssem0, ssem1)

    zeros = jnp.zeros((16,), jnp.float32)

    def zrow(r, carry):
        for k in range(D // 16):
            sbuf0[r, pl.ds(k * 16, 16)] = zeros
        return carry

    lax.fori_loop(0, B, zrow, 0)
    for j in range(ROWS // B):
        pltpu.sync_copy(sbuf0, acc_sh.at[pl.ds(base_r + j * B, B)])
    rem = ROWS % B
    pltpu.sync_copy(sbuf0.at[pl.ds(0, rem)],
                    acc_sh.at[pl.ds(base_r + (ROWS // B) * B, rem)])
    plsc.subcore_barrier()

    def compute(j):
        gb = gbufs[j]
        sb = sbufs[j]

        def edge(e4, cc):
            for u in range(4):
                e = e4 * 4 + u
                for k in range(DH // 16):
                    v = gb[e, pl.ds(c * DH + k * 16, 16)]
                    m = jnp.maximum(v, 0.0) + EPS
                    w = jnp.exp(m)
                    sb[e, pl.ds(k * 16, 16)] = w
                    sb[e, pl.ds(DH + k * 16, 16)] = m * w
            return cc

        lax.fori_loop(0, B // 4, edge, 0)

    def batch(g, j):
        pltpu.make_async_copy(x_hbm.at[sidx.at[g]], gbufs[j], gsems[j]).wait()

        @pl.when(g >= 2)
        def _():
            pltpu.make_async_copy(sbufs[j], acc_sh.at[didx.at[g - 2]],
                                  ssems[j]).wait()

        compute(j)
        pltpu.async_copy(sbufs[j], acc_sh.at[didx.at[g]], ssems[j], add=True)

        @pl.when(g + 2 < CH)
        def _():
            pltpu.async_copy(x_hbm.at[sidx.at[g + 2]], gbufs[j], gsems[j])

    def chunk(ch, carry):
        pltpu.sync_copy(src_hbm.at[s, pl.ds(ch * CH, CH)], sidx)
        pltpu.sync_copy(dst_hbm.at[s, pl.ds(ch * CH, CH)], didx)

        pltpu.async_copy(x_hbm.at[sidx.at[0]], gbufs[0], gsems[0])
        pltpu.async_copy(x_hbm.at[sidx.at[1]], gbufs[1], gsems[1])

        def pair(i, cc):
            batch(2 * i, 0)
            batch(2 * i + 1, 1)
            return cc

        lax.fori_loop(0, CH // 2, pair, 0)

        pltpu.make_async_copy(sbufs[0], acc_sh.at[didx.at[CH - 2]],
                              ssems[0]).wait()
        pltpu.make_async_copy(sbufs[1], acc_sh.at[didx.at[CH - 1]],
                              ssems[1]).wait()
        return carry

    lax.fori_loop(0, NCH, chunk, 0)
    plsc.subcore_barrier()

    pltpu.sync_copy(acc_sh.at[pl.ds(base_r, ROWS)],
                    acc_hbm.at[c, pl.ds(base_r, ROWS)])


_sc_agg = functools.partial(
    pl.kernel,
    out_type=jax.ShapeDtypeStruct((NCORE, NPAD, D), jnp.float32),
    mesh=plsc.VectorSubcoreMesh(core_axis_name="c", subcore_axis_name="s",
                                num_cores=NCORE),
    scratch_types=[
        pltpu.VMEM_SHARED((NPAD, D), jnp.float32),
        pltpu.VMEM((CH, B), jnp.int32),
        pltpu.VMEM((CH, B), jnp.int32),
        pltpu.VMEM((B, D), jnp.float32),
        pltpu.VMEM((B, D), jnp.float32),
        pltpu.VMEM((B, D), jnp.float32),
        pltpu.VMEM((B, D), jnp.float32),
        pltpu.SemaphoreType.DMA,
        pltpu.SemaphoreType.DMA,
        pltpu.SemaphoreType.DMA,
        pltpu.SemaphoreType.DMA,
    ],
)(_sc_agg_body)


def _tc_body(x_ref, acc_ref, w1_ref, b1_ref, g_ref, be_ref, w2_ref, b2_ref,
             o_ref):
    x = x_ref[...]
    a0 = acc_ref[0, :N, :]
    a1 = acc_ref[1, :N, :]
    denom = jnp.concatenate([a0[:, :DH], a1[:, :DH]], axis=1)
    numer = jnp.concatenate([a0[:, DH:], a1[:, DH:]], axis=1)
    out = numer / (denom + 1e-16) + x
    h = jnp.dot(out, w1_ref[...], preferred_element_type=jnp.float32)
    h = h + b1_ref[...]
    mean = jnp.mean(h, axis=0, keepdims=True)
    var = jnp.mean((h - mean) ** 2, axis=0, keepdims=True)
    hn = (h - mean) * lax.rsqrt(var + BN_EPS) * g_ref[...] + be_ref[...]
    hn = jnp.maximum(hn, 0.0)
    y = jnp.dot(hn, w2_ref[...], preferred_element_type=jnp.float32)
    y = y + b2_ref[...]
    o_ref[...] = x + jnp.maximum(y, 0.0)


def kernel(x, edge_index, W1, b1, gamma, beta, W2, b2):
    ei = edge_index.astype(jnp.int32)
    pad = NB * B - EPT
    src = jnp.pad(ei[0].reshape(NSUB, EPT), ((0, 0), (0, pad)),
                  constant_values=0).reshape(NSUB, NB, B)
    dst = jnp.pad(ei[1].reshape(NSUB, EPT), ((0, 0), (0, pad)),
                  constant_values=NPAD - 1).reshape(NSUB, NB, B)
    acc = _sc_agg(x, src, dst)
    return pl.pallas_call(
        _tc_body,
        out_shape=jax.ShapeDtypeStruct((N, D), jnp.float32),
    )(x, acc, W1, b1[None, :], gamma[None, :], beta[None, :], W2, b2[None, :])
